# Initial kernel scaffold; baseline (speedup 1.0000x reference)
import jax
import jax.numpy as jnp
from jax.experimental import pallas as pl


def kernel(x, g1, b1, wqkv, bqkv, wout, bout, g2, b2, wfc1, bfc1, wfc2, bfc2):
    raise NotImplementedError("write your pallas kernel here")


# speedup vs baseline: 1.2593x; 1.2593x over previous
import jax
import jax.numpy as jnp
from jax.experimental import pallas as pl
from jax.experimental.pallas import tpu as pltpu

EPS = 1e-5


def _pick_tb(B):
    for tb in (16, 8, 4, 2, 1):
        if B % tb == 0 and B // tb >= 2:
            return tb
    return B


def _make_body(C, H, P, N, TB):
    PN = P * N
    O = 2 * C + 1
    inv_cnt = 1.0 / (C * PN)
    NEG = -1e30
    f32, bf16 = jnp.float32, jnp.bfloat16

    def body(x_ref, vecs_ref, wqkv_ref, wout_ref, wfc1_ref, wfc2_ref, o_ref):
        x = x_ref[...].astype(f32)

        V = vecs_ref[...]
        g1, b1 = V[0:C, 0:1], V[0:C, 1:2]
        bqkv = V[0:O, 2:3]
        bout = V[0:C, 3:4]
        g2, b2 = V[0:C, 4:5], V[0:C, 5:6]
        bfc1 = V[0:H, 6:7]
        bfc2 = V[0:C, 7:8]

        seg_id = jax.lax.broadcasted_iota(jnp.int32, (1, 1, PN), 2) // N

        def gnorm(t, g, b):
            col = jnp.sum(t, axis=1, keepdims=True)
            mu = jnp.sum(col, axis=2, keepdims=True) * inv_cnt
            tc = t - mu
            sq = jnp.sum(tc * tc, axis=1, keepdims=True)
            var = jnp.sum(sq, axis=2, keepdims=True) * inv_cnt
            rstd = jax.lax.rsqrt(var + EPS)
            return tc * rstd * g + b

        def bdot(a, b_):
            return jax.lax.dot_general(a, b_, (((2,), (1,)), ((0,), (0,))),
                                       preferred_element_type=f32)

        def conv(w_ref, act_f32, bias):
            w = w_ref[...]
            wb = jnp.broadcast_to(w[None], (TB,) + w.shape)
            return bdot(wb, act_f32.astype(bf16)) + bias

        y1 = gnorm(x, g1, b1)
        qkv = conv(wqkv_ref, y1, bqkv)
        k = qkv[:, 0:C]
        v = qkv[:, C:2 * C]
        q = qkv[:, 2 * C:2 * C + 1]

        shift = jnp.zeros_like(q)
        dens = []
        for p in range(P):
            m = seg_id == p
            pmax = jnp.max(jnp.where(m, q, NEG), axis=2, keepdims=True)
            shift = jnp.where(m, pmax, shift)
        e = jnp.exp(q - shift)
        for p in range(P):
            m = seg_id == p
            dens.append(jnp.sum(jnp.where(m, e, 0.0), axis=2, keepdims=True))
        den = jnp.concatenate(dens, axis=2)

        segk = (jax.lax.broadcasted_iota(jnp.int32, (PN, P), 0) // N
                == jax.lax.broadcasted_iota(jnp.int32, (PN, P), 1)).astype(bf16)
        segq = (jax.lax.broadcasted_iota(jnp.int32, (P, PN), 1) // N
                == jax.lax.broadcasted_iota(jnp.int32, (P, PN), 0)).astype(bf16)
        ke = (k * e).astype(bf16)
        sums = bdot(ke, jnp.broadcast_to(segk[None], (TB, PN, P)))
        ctx = sums * pl.reciprocal(den, approx=True)
        ctx_full = bdot(ctx.astype(bf16),
                        jnp.broadcast_to(segq[None], (TB, P, PN)))

        attn = conv(wout_ref, jnp.maximum(v, 0.0) * ctx_full, bout)
        x2 = x + attn

        y2 = gnorm(x2, g2, b2)
        h = conv(wfc1_ref, y2, bfc1)
        h = h * jax.nn.sigmoid(h)
        mlp = conv(wfc2_ref, h, bfc2)
        o_ref[...] = (x2 + mlp).astype(o_ref.dtype)

    return body


def kernel(x, g1, b1, wqkv, bqkv, wout, bout, g2, b2, wfc1, bfc1, wfc2, bfc2):
    B, C, P, N = x.shape
    PN = P * N
    H = wfc1.shape[0]
    O = 2 * C + 1

    w_r = jnp.concatenate([wqkv[1:1 + C], wqkv[1 + C:], wqkv[0:1]], axis=0)
    b_r = jnp.concatenate([bqkv[1:1 + C], bqkv[1 + C:], bqkv[0:1]], axis=0)

    Rmax = max(O, H, C)

    def padcol(a):
        a = a.astype(jnp.float32).reshape(-1)
        return jnp.pad(a, (0, Rmax - a.shape[0]))

    vecs = jnp.stack([padcol(g1), padcol(b1), padcol(b_r), padcol(bout),
                      padcol(g2), padcol(b2), padcol(bfc1), padcol(bfc2)],
                     axis=1)

    bf16 = lambda a: a.astype(jnp.bfloat16)
    wqkv_b, wout_b = bf16(w_r), bf16(wout)
    wfc1_b, wfc2_b = bf16(wfc1), bf16(wfc2)

    TB = _pick_tb(B)
    G = B // TB
    x3 = x.reshape(B, C, PN)

    full = lambda a: pl.BlockSpec(a.shape, lambda i: (0,) * a.ndim)

    out = pl.pallas_call(
        _make_body(C, H, P, N, TB),
        out_shape=jax.ShapeDtypeStruct((B, C, PN), x.dtype),
        grid=(G,),
        in_specs=[pl.BlockSpec((TB, C, PN), lambda i: (i, 0, 0)),
                  full(vecs), full(wqkv_b), full(wout_b),
                  full(wfc1_b), full(wfc2_b)],
        out_specs=pl.BlockSpec((TB, C, PN), lambda i: (i, 0, 0)),
        compiler_params=pltpu.CompilerParams(
            dimension_semantics=("parallel",),
            vmem_limit_bytes=48 * 1024 * 1024),
    )(x3, vecs, wqkv_b, wout_b, wfc1_b, wfc2_b)

    return out.reshape(B, C, P, N)
